# baseline (speedup 1.0000x reference)
import functools

import jax
import jax.numpy as jnp
from jax.experimental import pallas as pl
from jax.experimental.pallas import tpu as pltpu

_N = 4096
_D = 128
_K = 3
_BLK = 256
_GRID = _N // _BLK
_SUB = 32
_NCHUNK = _N // 128
_LN2 = 0.6931471805599453
_FILL = 1.0e7
_BIG = 3.0e7


def _digamma(x):
    s = (1.0 / x + 1.0 / (x + 1.0) + 1.0 / (x + 2.0)
         + 1.0 / (x + 3.0) + 1.0 / (x + 4.0) + 1.0 / (x + 5.0))
    z = x + 6.0
    zi = 1.0 / z
    zi2 = zi * zi
    psi_z = jnp.log(z) - 0.5 * zi - zi2 * (
        1.0 / 12.0 - zi2 * (1.0 / 120.0 - zi2 * (1.0 / 252.0)))
    return psi_z - s


def _merge_sorted4(r, s):
    r1, r2, r3, r4 = r
    s1, s2, s3, s4 = s
    z1 = jnp.minimum(r1, s4)
    z2 = jnp.minimum(r2, s3)
    z3 = jnp.minimum(r3, s2)
    z4 = jnp.minimum(r4, s1)
    a1 = jnp.minimum(z1, z3)
    a3 = jnp.maximum(z1, z3)
    a2 = jnp.minimum(z2, z4)
    a4 = jnp.maximum(z2, z4)
    return (jnp.minimum(a1, a2), jnp.maximum(a1, a2),
            jnp.minimum(a3, a4), jnp.maximum(a3, a4))


def _mi_kernel(yr_ref, x_ref, y_ref, o_ref, xn_ref, d_ref):
    i = pl.program_id(0)

    y_full = y_ref[...]

    @pl.when(i == 0)
    def _():
        x_full = x_ref[...]
        nrm = jnp.maximum(jnp.sqrt(jnp.sum(x_full * x_full, axis=1,
                                           keepdims=True)), 1e-8)
        xn_ref[...] = x_full / nrm

    xn_full = xn_ref[...]
    xn_rows = xn_ref[pl.ds(i * _BLK, _BLK), :]

    sim = jax.lax.dot_general(xn_rows, xn_full,
                              (((1,), (1,)), ((), ())),
                              preferred_element_type=jnp.float32)
    d_ref[...] = 1.0 - sim
    sub = d_ref[:, pl.ds(i * _BLK, _BLK)]
    lr = jax.lax.broadcasted_iota(jnp.int32, (_BLK, _BLK), 0)
    lc = jax.lax.broadcasted_iota(jnp.int32, (_BLK, _BLK), 1)
    d_ref[:, pl.ds(i * _BLK, _BLK)] = jnp.where(lr == lc, 0.0, sub)

    anchors = []
    for rt in range(_BLK // _SUB):
        yr = yr_ref[pl.ds(rt * _SUB, _SUB), :]

        def body(c, carry):
            r1, r2, r3, r4 = carry
            d = d_ref[pl.ds(rt * _SUB, _SUB), pl.ds(c * 128, 128)]
            yc = y_ref[:, pl.ds(c * 128, 128)]
            v = jnp.where(yr == yc, d, _FILL)
            n1 = jnp.minimum(r1, v)
            t1 = jnp.maximum(r1, v)
            n2 = jnp.minimum(r2, t1)
            t2 = jnp.maximum(r2, t1)
            n3 = jnp.minimum(r3, t2)
            t3 = jnp.maximum(r3, t2)
            n4 = jnp.minimum(r4, t3)
            return (n1, n2, n3, n4)

        init = tuple(jnp.full((_SUB, 128), _BIG, dtype=jnp.float32)
                     for _ in range(4))
        r = jax.lax.fori_loop(0, _NCHUNK, body, init, unroll=True)
        for s in (1, 2, 4, 8, 16, 32, 64):
            shifted = tuple(pltpu.roll(t, 128 - s, 1) for t in r)
            r = _merge_sorted4(r, shifted)
        anchors.append(r[3][:, 0:1])

    anchor = jnp.concatenate(anchors, axis=0)

    dists = d_ref[...]
    cnt = jnp.sum((dists <= anchor).astype(jnp.int32), axis=1,
                  keepdims=True) - 1
    part = jnp.sum(_digamma(cnt.astype(jnp.float32)), keepdims=True)

    @pl.when(i == 0)
    def _():
        o_ref[...] = jnp.zeros_like(o_ref)

    o_ref[...] += part

    @pl.when(i == _GRID - 1)
    def _():
        acc = o_ref[...]
        n = jnp.float32(_N)
        n1 = jnp.sum(y_full, keepdims=True).astype(jnp.float32)
        n0 = n - n1
        avg_nx = (n0 / n) * _digamma(n0) + (n1 / n) * _digamma(n1)
        mi = _digamma(n) - avg_nx + _digamma(jnp.float32(_K)) - acc / n
        o_ref[...] = mi / _LN2


@jax.jit
def kernel(X, y):
    y_row = y.reshape(1, _N)
    y_col = y.reshape(_N, 1)
    out = pl.pallas_call(
        _mi_kernel,
        grid=(_GRID,),
        in_specs=[
            pl.BlockSpec((_BLK, 1), lambda i: (i, 0)),
            pl.BlockSpec((_N, _D), lambda i: (0, 0)),
            pl.BlockSpec((1, _N), lambda i: (0, 0)),
        ],
        out_specs=pl.BlockSpec((1, 1), lambda i: (0, 0)),
        out_shape=jax.ShapeDtypeStruct((1, 1), jnp.float32),
        scratch_shapes=[pltpu.VMEM((_N, _D), jnp.float32),
                        pltpu.VMEM((_BLK, _N), jnp.float32)],
    )(y_col, X, y_row)
    return out[0, 0]


# speedup vs baseline: 1.2425x; 1.2425x over previous
import functools

import jax
import jax.numpy as jnp
from jax.experimental import pallas as pl
from jax.experimental.pallas import tpu as pltpu

_N = 4096
_D = 128
_K = 3
_BLK = 256
_GRID = _N // _BLK
_SUB = 32
_NCHUNK = _N // 128
_LN2 = 0.6931471805599453
_FILL = 1.0e7
_BIG = 3.0e7


def _digamma(x):
    s = (1.0 / x + 1.0 / (x + 1.0) + 1.0 / (x + 2.0)
         + 1.0 / (x + 3.0) + 1.0 / (x + 4.0) + 1.0 / (x + 5.0))
    z = x + 6.0
    zi = 1.0 / z
    zi2 = zi * zi
    psi_z = jnp.log(z) - 0.5 * zi - zi2 * (
        1.0 / 12.0 - zi2 * (1.0 / 120.0 - zi2 * (1.0 / 252.0)))
    return psi_z - s


def _anchor_of_candidates(cand):
    work = cand
    remaining = jnp.full((_SUB, 1), _K + 1, dtype=jnp.int32)
    anchor = jnp.zeros((_SUB, 1), dtype=jnp.float32)
    for _ in range(_K + 1):
        m = jnp.min(work, axis=1, keepdims=True)
        hit = work == m
        c = jnp.sum(hit.astype(jnp.int32), axis=1, keepdims=True)
        anchor = jnp.where(remaining > 0, m, anchor)
        remaining = remaining - c
        work = jnp.where(hit, _BIG, work)
    return anchor


def _mi_kernel(yr_ref, x_ref, y_ref, o_ref, xn_ref, d_ref):
    i = pl.program_id(0)

    y_full = y_ref[...]

    @pl.when(i == 0)
    def _():
        x_full = x_ref[...]
        nrm = jnp.maximum(jnp.sqrt(jnp.sum(x_full * x_full, axis=1,
                                           keepdims=True)), 1e-8)
        xn_ref[...] = x_full / nrm

    xn_full = xn_ref[...]
    xn_rows = xn_ref[pl.ds(i * _BLK, _BLK), :]

    sim = jax.lax.dot_general(xn_rows, xn_full,
                              (((1,), (1,)), ((), ())),
                              preferred_element_type=jnp.float32)
    d_ref[...] = 1.0 - sim
    sub = d_ref[:, pl.ds(i * _BLK, _BLK)]
    lr = jax.lax.broadcasted_iota(jnp.int32, (_BLK, _BLK), 0)
    lc = jax.lax.broadcasted_iota(jnp.int32, (_BLK, _BLK), 1)
    d_ref[:, pl.ds(i * _BLK, _BLK)] = jnp.where(lr == lc, 0.0, sub)

    anchors = []
    for rt in range(_BLK // _SUB):
        yr = yr_ref[pl.ds(rt * _SUB, _SUB), :]

        def body(c, carry):
            r1, r2, r3, r4 = carry
            d = d_ref[pl.ds(rt * _SUB, _SUB), pl.ds(c * 128, 128)]
            yc = y_ref[:, pl.ds(c * 128, 128)]
            v = jnp.where(yr == yc, d, _FILL)
            n1 = jnp.minimum(r1, v)
            t1 = jnp.maximum(r1, v)
            n2 = jnp.minimum(r2, t1)
            t2 = jnp.maximum(r2, t1)
            n3 = jnp.minimum(r3, t2)
            t3 = jnp.maximum(r3, t2)
            n4 = jnp.minimum(r4, t3)
            return (n1, n2, n3, n4)

        init = tuple(jnp.full((_SUB, 128), _BIG, dtype=jnp.float32)
                     for _ in range(4))
        r1, r2, r3, r4 = jax.lax.fori_loop(0, _NCHUNK, body, init,
                                           unroll=True)
        cand = jnp.concatenate([r1, r2, r3, r4], axis=1)
        anchors.append(_anchor_of_candidates(cand))

    anchor = jnp.concatenate(anchors, axis=0)

    dists = d_ref[...]
    cnt = jnp.sum((dists <= anchor).astype(jnp.int32), axis=1,
                  keepdims=True) - 1
    part = jnp.sum(_digamma(cnt.astype(jnp.float32)), keepdims=True)

    @pl.when(i == 0)
    def _():
        o_ref[...] = jnp.zeros_like(o_ref)

    o_ref[...] += part

    @pl.when(i == _GRID - 1)
    def _():
        acc = o_ref[...]
        n = jnp.float32(_N)
        n1 = jnp.sum(y_full, keepdims=True).astype(jnp.float32)
        n0 = n - n1
        avg_nx = (n0 / n) * _digamma(n0) + (n1 / n) * _digamma(n1)
        mi = _digamma(n) - avg_nx + _digamma(jnp.float32(_K)) - acc / n
        o_ref[...] = mi / _LN2


@jax.jit
def kernel(X, y):
    y_row = y.reshape(1, _N)
    y_col = y.reshape(_N, 1)
    out = pl.pallas_call(
        _mi_kernel,
        grid=(_GRID,),
        in_specs=[
            pl.BlockSpec((_BLK, 1), lambda i: (i, 0)),
            pl.BlockSpec((_N, _D), lambda i: (0, 0)),
            pl.BlockSpec((1, _N), lambda i: (0, 0)),
        ],
        out_specs=pl.BlockSpec((1, 1), lambda i: (0, 0)),
        out_shape=jax.ShapeDtypeStruct((1, 1), jnp.float32),
        scratch_shapes=[pltpu.VMEM((_N, _D), jnp.float32),
                        pltpu.VMEM((_BLK, _N), jnp.float32)],
    )(y_col, X, y_row)
    return out[0, 0]
